# baseline (speedup 1.0000x reference)
import functools

import jax
import jax.numpy as jnp
from jax import lax
from jax.experimental import pallas as pl
from jax.experimental.pallas import tpu as pltpu
from jax.experimental.pallas import tpu_sc as plsc

_PERM = (0, 2, 1, 3)
_NROWS = len(_PERM)
_D = 128
_LANES = 16

_mesh = plsc.VectorSubcoreMesh(core_axis_name="c", subcore_axis_name="s")


@functools.partial(
    pl.kernel,
    mesh=_mesh,
    out_type=jax.ShapeDtypeStruct((_NROWS, _D), jnp.float32),
    scratch_types=[
        pltpu.VMEM((_NROWS, _D), jnp.float32),
        pltpu.VMEM((_NROWS, _D), jnp.float32),
    ],
)
def _gather_rows(x_hbm, out_hbm, buf_in, buf_out):
    c = lax.axis_index("c")
    s = lax.axis_index("s")

    @pl.when(jnp.logical_and(c == 0, s == 0))
    def _():
        pltpu.sync_copy(x_hbm.at[pl.ds(0, _NROWS)], buf_in)
        for i, src in enumerate(_PERM):
            for j in range(_D // _LANES):
                sl = pl.ds(j * _LANES, _LANES)
                buf_out[i, sl] = buf_in[src, sl]
        pltpu.sync_copy(buf_out, out_hbm)


def kernel(x):
    return _gather_rows(x).reshape(2, 2, _D)


# speedup vs baseline: 1.0960x; 1.0960x over previous
import functools

import jax
import jax.numpy as jnp
from jax import lax
from jax.experimental import pallas as pl
from jax.experimental.pallas import tpu as pltpu
from jax.experimental.pallas import tpu_sc as plsc

_PERM = (0, 2, 1, 3)
_D = 128

_mesh = plsc.ScalarSubcoreMesh(axis_name="c", num_cores=2)


@functools.partial(
    pl.kernel,
    mesh=_mesh,
    out_type=jax.ShapeDtypeStruct((2, 2, _D), jnp.float32),
    scratch_types=[pltpu.SemaphoreType.DMA],
)
def _gather_rows(x_hbm, out_hbm, sem):
    c = lax.axis_index("c")

    @pl.when(c == 0)
    def _():
        copies = [
            pltpu.make_async_copy(
                x_hbm.at[src], out_hbm.at[i // 2, i % 2], sem
            )
            for i, src in enumerate(_PERM)
        ]
        for cp in copies:
            cp.start()
        for cp in copies:
            cp.wait()


def kernel(x):
    return _gather_rows(x)


# speedup vs baseline: 1.0998x; 1.0035x over previous
import functools

import jax
import jax.numpy as jnp
from jax import lax
from jax.experimental import pallas as pl
from jax.experimental.pallas import tpu as pltpu
from jax.experimental.pallas import tpu_sc as plsc

_PERM = (0, 2, 1, 3)
_D = 128

_mesh = plsc.ScalarSubcoreMesh(axis_name="c", num_cores=2)


@functools.partial(
    pl.kernel,
    mesh=_mesh,
    out_type=jax.ShapeDtypeStruct((2, 2, _D), jnp.float32),
    scratch_types=[pltpu.SemaphoreType.DMA],
)
def _gather_rows(x_hbm, out_hbm, sem):
    c = lax.axis_index("c")

    for core in (0, 1):
        @pl.when(c == core)
        def _(core=core):
            copies = [
                pltpu.make_async_copy(
                    x_hbm.at[src], out_hbm.at[i // 2, i % 2], sem
                )
                for i, src in enumerate(_PERM)
                if i % 2 == core
            ]
            for cp in copies:
                cp.start()
            for cp in copies:
                cp.wait()


def kernel(x):
    return _gather_rows(x)


# speedup vs baseline: 1.1895x; 1.0815x over previous
import functools

import jax
import jax.numpy as jnp
from jax import lax
from jax.experimental import pallas as pl
from jax.experimental.pallas import tpu as pltpu
from jax.experimental.pallas import tpu_sc as plsc

_PERM = (0, 2, 1, 3)
_D = 128

_mesh = plsc.ScalarSubcoreMesh(axis_name="c", num_cores=1)


@functools.partial(
    pl.kernel,
    mesh=_mesh,
    out_type=jax.ShapeDtypeStruct((2, 2, _D), jnp.float32),
    scratch_types=[pltpu.SemaphoreType.DMA],
)
def _gather_rows(x_hbm, out_hbm, sem):
    copies = [
        pltpu.make_async_copy(x_hbm.at[src], out_hbm.at[i // 2, i % 2], sem)
        for i, src in enumerate(_PERM)
    ]
    for cp in copies:
        cp.start()
    for cp in copies:
        cp.wait()


def kernel(x):
    return _gather_rows(x)
